# baseline (speedup 1.0000x reference)
import jax
import jax.numpy as jnp
from jax import lax
from jax.experimental import pallas as pl
from jax.experimental.pallas import tpu as pltpu

_E = 16
_BT = 512
_NBUF = 6


def _gate_kernel(h_hbm, w_ref, idx_ref, wt_ref, hbuf, sem):
    i = pl.program_id(0)
    nblk = pl.num_programs(0)

    def copy(j, slot):
        return pltpu.make_async_copy(
            h_hbm.at[pl.ds(j * _BT, _BT), :], hbuf.at[slot], sem.at[slot])

    @pl.when(i == 0)
    def _():
        for j in range(_NBUF - 1):
            copy(j, j).start()

    nxt = i + _NBUF - 1

    @pl.when(nxt < nblk)
    def _():
        copy(nxt, lax.rem(nxt, _NBUF)).start()

    slot = lax.rem(i, _NBUF)
    copy(i, slot).wait()
    logits = lax.dot_general(w_ref[...], hbuf[slot], (((1,), (1,)), ((), ())),
                             preferred_element_type=jnp.float32)
    sub = lax.broadcasted_iota(jnp.int32, logits.shape, 0)
    m1 = jnp.max(logits, axis=0, keepdims=True)
    i1 = jnp.min(jnp.where(logits == m1, sub, _E), axis=0, keepdims=True)
    masked = jnp.where(sub == i1, -jnp.inf, logits)
    m2 = jnp.max(masked, axis=0, keepdims=True)
    i2 = jnp.min(jnp.where(masked == m2, sub, _E), axis=0, keepdims=True)
    e2 = jnp.exp(m2 - m1)
    denom = 1.0 + e2
    idx_ref[...] = jnp.concatenate([i1, i2], axis=0)
    wt_ref[...] = jnp.concatenate([1.0 / denom, e2 / denom], axis=0)


def kernel(hidden_states, weight):
    bsz, seq_len, dim = hidden_states.shape
    h = hidden_states.reshape(-1, dim)
    tokens = h.shape[0]
    nblk = tokens // _BT
    idx_t, wt_t = pl.pallas_call(
        _gate_kernel,
        grid=(nblk,),
        in_specs=[
            pl.BlockSpec(memory_space=pl.ANY),
            pl.BlockSpec((_E, dim), lambda i: (0, 0)),
        ],
        out_specs=[
            pl.BlockSpec((2, _BT), lambda i: (0, i)),
            pl.BlockSpec((2, _BT), lambda i: (0, i)),
        ],
        out_shape=[
            jax.ShapeDtypeStruct((2, tokens), jnp.int32),
            jax.ShapeDtypeStruct((2, tokens), jnp.float32),
        ],
        scratch_shapes=[
            pltpu.VMEM((_NBUF, _BT, dim), jnp.float32),
            pltpu.SemaphoreType.DMA((_NBUF,)),
        ],
        compiler_params=pltpu.CompilerParams(
            dimension_semantics=("arbitrary",)),
    )(h, weight)
    return (idx_t.T, wt_t.T, jnp.float32(0.0))
